# baseline (speedup 1.0000x reference)
import jax
import jax.numpy as jnp
from jax.experimental import pallas as pl

_C, _H, _W = 256, 64, 64
_HALF = _C // 2


def _body(rw_ref, cw_ref, out_ref):
    cw_t = cw_ref[:_W, :].T
    rw_t = rw_ref[:_H, :].T
    out_ref[0, :_HALF, :, :] = jnp.broadcast_to(cw_t[:, None, :], (_HALF, _H, _W))
    out_ref[0, _HALF:, :, :] = jnp.broadcast_to(rw_t[:, :, None], (_HALF, _H, _W))


def kernel(x, row_weight, col_weight):
    b = x.shape[0]
    return pl.pallas_call(
        _body,
        grid=(b,),
        in_specs=[
            pl.BlockSpec((128, 128), lambda i: (0, 0)),
            pl.BlockSpec((128, 128), lambda i: (0, 0)),
        ],
        out_specs=pl.BlockSpec((1, _C, _H, _W), lambda i: (i, 0, 0, 0)),
        out_shape=jax.ShapeDtypeStruct((b, _C, _H, _W), jnp.float32),
    )(row_weight, col_weight)


# speedup vs baseline: 1.7352x; 1.7352x over previous
import jax
import jax.numpy as jnp
from jax.experimental import pallas as pl
from jax.experimental.pallas import tpu as pltpu

_C, _H, _W = 256, 64, 64
_HALF = _C // 2
_HW = _H * _W


def _body(rw_ref, cw_ref, out_ref, pos_ref):
    @pl.when(pl.program_id(0) == 0)
    def _init():
        cw_t = cw_ref[:_W, :].T
        rw_t = rw_ref[:_H, :].T
        pos_ref[:_HALF, :] = jnp.broadcast_to(
            cw_t[:, None, :], (_HALF, _H, _W)).reshape(_HALF, _HW)
        pos_ref[_HALF:, :] = jnp.broadcast_to(
            rw_t[:, :, None], (_HALF, _H, _W)).reshape(_HALF, _HW)

    out_ref[0] = pos_ref[...]


def kernel(x, row_weight, col_weight):
    b = x.shape[0]
    out = pl.pallas_call(
        _body,
        grid=(b,),
        in_specs=[
            pl.BlockSpec((128, 128), lambda i: (0, 0)),
            pl.BlockSpec((128, 128), lambda i: (0, 0)),
        ],
        out_specs=pl.BlockSpec((1, _C, _HW), lambda i: (i, 0, 0)),
        out_shape=jax.ShapeDtypeStruct((b, _C, _HW), jnp.float32),
        scratch_shapes=[pltpu.VMEM((_C, _HW), jnp.float32)],
    )(row_weight, col_weight)
    return out.reshape(b, _C, _H, _W)


# speedup vs baseline: 1.7630x; 1.0160x over previous
import jax
import jax.numpy as jnp
from jax.experimental import pallas as pl
from jax.experimental.pallas import tpu as pltpu

_C, _H, _W = 256, 64, 64
_HALF = _C // 2
_HW = _H * _W


def _make_body(b):
    def _body(rw_ref, cw_ref, out_ref, pos_ref, sem):
        cw_t = cw_ref[:_W, :].T
        rw_t = rw_ref[:_H, :].T
        pos_ref[:_HALF, :] = jnp.broadcast_to(
            cw_t[:, None, :], (_HALF, _H, _W)).reshape(_HALF, _HW)
        pos_ref[_HALF:, :] = jnp.broadcast_to(
            rw_t[:, :, None], (_HALF, _H, _W)).reshape(_HALF, _HW)
        copies = [
            pltpu.make_async_copy(pos_ref, out_ref.at[i], sem)
            for i in range(b)
        ]
        for cp in copies:
            cp.start()
        for cp in copies:
            cp.wait()
    return _body


def kernel(x, row_weight, col_weight):
    b = x.shape[0]
    out = pl.pallas_call(
        _make_body(b),
        in_specs=[
            pl.BlockSpec(memory_space=pltpu.VMEM),
            pl.BlockSpec(memory_space=pltpu.VMEM),
        ],
        out_specs=pl.BlockSpec(memory_space=pl.ANY),
        out_shape=jax.ShapeDtypeStruct((b, _C, _HW), jnp.float32),
        scratch_shapes=[
            pltpu.VMEM((_C, _HW), jnp.float32),
            pltpu.SemaphoreType.DMA,
        ],
    )(row_weight, col_weight)
    return out.reshape(b, _C, _H, _W)
